# baseline (speedup 1.0000x reference)
import jax
import jax.numpy as jnp
from jax.experimental import pallas as pl
from jax.experimental.pallas import tpu as pltpu

N, F, D = 524288, 26, 4
LANES = 128
NT = N // LANES
BNT = 256
BN = BNT * LANES


CH = 16


def _lvr_block(x_ref, idx_ref, o_ref):
    def body(c, carry):
        sl = pl.ds(c * CH * LANES, CH * LANES)
        x3 = x_ref[:, sl].reshape(F, CH, LANES)
        i3 = idx_ref[:, sl].reshape(F, CH, LANES)
        base = c * CH * D
        for k in range(D):
            if k == 0:
                plane = jnp.where(i3 > 0, 1.0, x3)
            elif k == D - 1:
                plane = jnp.where(i3 < D - 1, 0.0, x3)
            else:
                plane = jnp.where(i3 > k, 1.0, jnp.where(i3 < k, 0.0, x3))
            o_ref[:, pl.Slice(base + k, CH, D), :] = plane
        return carry
    jax.lax.fori_loop(0, BNT // CH, body, 0)


def kernel(x, indices):
    out = pl.pallas_call(
        _lvr_block,
        grid=(NT // BNT,),
        in_specs=[
            pl.BlockSpec((F, BN), lambda i: (0, i)),
            pl.BlockSpec((F, BN), lambda i: (0, i)),
        ],
        out_specs=pl.BlockSpec((F, D * BNT, LANES), lambda i: (0, i, 0)),
        out_shape=jax.ShapeDtypeStruct((F, D * NT, LANES), jnp.float32),
        compiler_params=pltpu.CompilerParams(
            dimension_semantics=("parallel",)),
    )(x.T, indices.T)
    return out.reshape(F, NT, D, LANES).transpose(1, 3, 0, 2).reshape(N, F, D)
